# baseline (speedup 1.0000x reference)
import jax
import jax.numpy as jnp
from jax import lax
from jax.experimental import pallas as pl
from jax.experimental.pallas import tpu as pltpu
from jax.experimental.pallas import tpu_sc as plsc

N = 64
HH = 512
ROWW = 1536
G = 32
NP = G * G
NUM_MASKED = 768
PR = 16
NC, NS = 2, 16
NW = NC * NS
CPW = (N * G) // NW
NB = 4


def _sc_body(x_hbm, idx_hbm, tok_hbm, out_hbm,
             xb0, xb1, xb2, xb3, tok_v, idx_v, mask_v, mlist_sm,
             isem0, isem1, isem2, isem3, osem0, osem1, osem2, osem3):
    wid = lax.axis_index("s") * NC + lax.axis_index("c")
    g0 = wid * CPW
    xbufs = (xb0, xb1, xb2, xb3)
    isems = (isem0, isem1, isem2, isem3)
    osems = (osem0, osem1, osem2, osem3)

    pltpu.sync_copy(tok_hbm, tok_v)

    zeros16 = jnp.zeros((16,), jnp.int32)
    ones16 = jnp.ones((16,), jnp.int32)

    CW = PR * ROWW

    def in_dma(f, q):
        return pltpu.make_async_copy(
            x_hbm.at[pl.ds((g0 + f) * CW, CW)], xbufs[q], isems[q])

    def out_dma(c, p):
        return pltpu.make_async_copy(
            xbufs[p], out_hbm.at[pl.ds((g0 + c) * CW, CW)], osems[p])

    in_dma(0, 0).start()
    in_dma(1, 1).start()

    def build_mask(s):
        b = 2 * wid + s
        pltpu.sync_copy(
            idx_hbm.at[pl.ds(8 * (b // 8), 8), pl.ds(NUM_MASKED, NP - NUM_MASKED)],
            idx_v)
        rb = b % 8

        def zbody(j, _):
            mask_v[pl.ds(j * 16, 16)] = zeros16
            return 0
        lax.fori_loop(0, NP // 16, zbody, 0)
        for j in range(16):
            iv = idx_v[rb, pl.ds(j * 16, 16)]
            plsc.store_scatter(mask_v, [iv], ones16)

    def do_chunk(t, j):
        c = 4 * t + j
        p = j
        if j == 0:
            @pl.when((t & 7) == 0)
            def _():
                build_mask(t >> 3)
        in_dma(c, p).wait()
        gh = c & (G - 1)
        base = gh * G
        m0 = mask_v[pl.ds(base, 16)]
        m1 = mask_v[pl.ds(base + 16, 16)]
        mlist_sm[0] = 0
        for gw in range(G):
            lane = m0[gw] if gw < 16 else m1[gw - 16]

            @pl.when(lane == 0)
            def _():
                cnt = mlist_sm[0]
                mlist_sm[1 + cnt] = gw
                mlist_sm[0] = cnt + 1
        xb = xbufs[p]

        def cbody(jj, _):
            c0 = mlist_sm[1 + jj] * 48
            for q in range(3):
                cq = c0 + q * 16
                for r in range(PR):
                    xb[pl.ds(r * ROWW + cq, 16)] = tok_v[pl.ds(r * ROWW + cq, 16)]
            return 0
        lax.fori_loop(0, mlist_sm[0], cbody, 0)
        out_dma(c, p).start()
        f = c + 2
        q = (j + 2) % NB

        def refill():
            in_dma(f, q).start()
        if j < 2:
            @pl.when(t > 0)
            def _():
                out_dma(f - NB, q).wait()
            refill()
        else:
            @pl.when(t < (CPW // 4) - 1)
            def _():
                out_dma(f - NB, q).wait()
                refill()

    def macro(t, _):
        for j in range(4):
            do_chunk(t, j)
        return 0
    lax.fori_loop(0, CPW // 4, macro, 0)

    for p in range(4):
        out_dma(CPW - 4 + p, p).wait()


def kernel(x, masked_token, shuffle_indices):
    x2 = x.reshape(N * HH * ROWW)
    idx2 = shuffle_indices.astype(jnp.int32)
    tok2 = jnp.tile(masked_token.reshape(16, 48), (1, G)).reshape(PR * ROWW)
    mesh = plsc.VectorSubcoreMesh(core_axis_name="c", subcore_axis_name="s",
                                  num_cores=NC, num_subcores=NS)
    f = pl.kernel(
        _sc_body,
        out_type=jax.ShapeDtypeStruct((N * HH * ROWW,), jnp.float32),
        mesh=mesh,
        compiler_params=pltpu.CompilerParams(use_tc_tiling_on_sc=True,
                                             needs_layout_passes=False),
        scratch_types=[
            pltpu.VMEM((PR * ROWW,), jnp.float32),
            pltpu.VMEM((PR * ROWW,), jnp.float32),
            pltpu.VMEM((PR * ROWW,), jnp.float32),
            pltpu.VMEM((PR * ROWW,), jnp.float32),
            pltpu.VMEM((PR * ROWW,), jnp.float32),
            pltpu.VMEM((8, NP - NUM_MASKED), jnp.int32),
            pltpu.VMEM((NP + 16,), jnp.int32),
            pltpu.SMEM((G + 2,), jnp.int32),
            pltpu.SemaphoreType.DMA,
            pltpu.SemaphoreType.DMA,
            pltpu.SemaphoreType.DMA,
            pltpu.SemaphoreType.DMA,
            pltpu.SemaphoreType.DMA,
            pltpu.SemaphoreType.DMA,
            pltpu.SemaphoreType.DMA,
            pltpu.SemaphoreType.DMA,
        ],
    )
    out = f(x2, idx2, tok2)
    return out.reshape(N, HH, HH, 3)


# speedup vs baseline: 40.9684x; 40.9684x over previous
import jax
import jax.numpy as jnp
from jax import lax
from jax.experimental import pallas as pl
from jax.experimental.pallas import tpu as pltpu
from jax.experimental.pallas import tpu_sc as plsc

N = 64
HH = 512
ROWW = 1536
G = 32
NP = G * G
NUM_MASKED = 768
NC, NS = 2, 16
NW = NC * NS


def _sc_mask_body(idx_hbm, mask_hbm, idx_v, mask_v):
    wid = lax.axis_index("s") * NC + lax.axis_index("c")
    zeros16 = jnp.zeros((16,), jnp.int32)
    ones16 = jnp.ones((16,), jnp.int32)

    for s in range(2):
        b = 2 * wid + s
        pltpu.sync_copy(
            idx_hbm.at[pl.ds(8 * (b // 8), 8),
                       pl.ds(NUM_MASKED, NP - NUM_MASKED)],
            idx_v)
        rb = b % 8
        for r in range(G):
            mask_v[r, pl.ds(0, 16)] = zeros16
            mask_v[r, pl.ds(16, 16)] = zeros16
        for j in range(16):
            iv = idx_v[rb, pl.ds(j * 16, 16)]
            plsc.store_scatter(mask_v, [iv >> 5, iv & (G - 1)], ones16)
        pltpu.sync_copy(mask_v, mask_hbm.at[b])


def _tc_select_body(mask_ref, x_ref, tok_ref, rrows_ref, rcols_ref, out_ref):
    u = mask_ref[0].astype(jnp.bfloat16)
    mrows = jax.lax.dot_general(
        rrows_ref[...], u, (((1,), (0,)), ((), ())),
        preferred_element_type=jnp.float32)
    m = jax.lax.dot_general(
        mrows.astype(jnp.bfloat16), rcols_ref[...], (((1,), (0,)), ((), ())),
        preferred_element_type=jnp.float32)
    out_ref[0] = jnp.where(m > 0.5, x_ref[0], tok_ref[...])


def kernel(x, masked_token, shuffle_indices):
    idx2 = shuffle_indices.astype(jnp.int32)

    sc_mesh = plsc.VectorSubcoreMesh(core_axis_name="c", subcore_axis_name="s",
                                     num_cores=NC, num_subcores=NS)
    sc_mask = pl.kernel(
        _sc_mask_body,
        out_type=jax.ShapeDtypeStruct((N, G, G), jnp.int32),
        mesh=sc_mesh,
        compiler_params=pltpu.CompilerParams(needs_layout_passes=False),
        scratch_types=[
            pltpu.VMEM((8, NP - NUM_MASKED), jnp.int32),
            pltpu.VMEM((G, G), jnp.int32),
        ],
    )
    mask = sc_mask(idx2)

    tok_full = jnp.tile(masked_token.reshape(16, 48), (G, G))
    rrows = (jnp.arange(HH, dtype=jnp.int32)[:, None] // 16
             == jnp.arange(G, dtype=jnp.int32)[None, :]).astype(jnp.bfloat16)
    rcols = (jnp.arange(ROWW, dtype=jnp.int32)[None, :] // 48
             == jnp.arange(G, dtype=jnp.int32)[:, None]).astype(jnp.bfloat16)

    x3 = x.reshape(N, HH, ROWW)
    out3 = pl.pallas_call(
        _tc_select_body,
        grid=(N,),
        in_specs=[
            pl.BlockSpec((1, G, G), lambda b: (b, 0, 0)),
            pl.BlockSpec((1, HH, ROWW), lambda b: (b, 0, 0)),
            pl.BlockSpec((HH, ROWW), lambda b: (0, 0)),
            pl.BlockSpec((HH, G), lambda b: (0, 0)),
            pl.BlockSpec((G, ROWW), lambda b: (0, 0)),
        ],
        out_specs=pl.BlockSpec((1, HH, ROWW), lambda b: (b, 0, 0)),
        out_shape=jax.ShapeDtypeStruct((N, HH, ROWW), jnp.float32),
    )(mask, x3, tok_full, rrows, rcols)
    return out3.reshape(N, HH, HH, 3)
